# baseline (speedup 1.0000x reference)
import functools

import jax
import jax.numpy as jnp
from jax import lax
from jax.experimental import pallas as pl
from jax.experimental.pallas import tpu as pltpu
from jax.experimental.pallas import tpu_sc as plsc

_D = _H = _W = 128
_HW = _H * _W
_DHW = _D * _HW
_C = 16
_L = 16
_CH = 128
_NC = 2
_NS = 16
_NW = _NC * _NS
_OFFS = (0, 1, _W, _W + 1, _HW, _HW + 1, _HW + _W, _HW + _W + 1)


def _axis_coords(v, dim):
    t = (v - 0.5) * 2.0
    i = (t + 1.0) * 0.5 * jnp.float32(dim - 1)
    i0 = jnp.minimum(i.astype(jnp.int32), dim - 2)
    w1 = i - i0.astype(jnp.float32)
    return i0, w1, 1.0 - w1


def _corner_base(vx, vy, vz):
    x0, _, _ = _axis_coords(vx, _W)
    y0, _, _ = _axis_coords(vy, _H)
    z0, _, _ = _axis_coords(vz, _D)
    return z0 * _HW + y0 * _W + x0


def _lerp_weights(vx, vy, vz):
    _, wx1, _ = _axis_coords(vx, _W)
    _, wy1, _ = _axis_coords(vy, _H)
    _, wz1, _ = _axis_coords(vz, _D)
    return wx1, wy1, wz1


def _sc_body(table, cx, cy, cz, out, *scr):
    cbuf = (scr[0:3], scr[3:6])
    vbuf = scr[6:8]
    rows = (scr[8:16], scr[16:24])
    obuf = scr[24:26]
    sem_g, sem_c, sem_o = scr[26:29]
    tview = [table.at[pl.ds(o, _DHW - _OFFS[-1])] for o in _OFFS]

    nper = cx.shape[0] // _NW
    chunks = nper // _CH
    wid = lax.axis_index("s") * _NC + lax.axis_index("c")
    cin = (cx, cy, cz)

    def base_of(s):
        return wid * nper + s * _CH

    def start_coords(s, b):
        return [pltpu.async_copy(cin[k].at[pl.ds(base_of(s), _CH)],
                                 cbuf[b][k], sem_c) for k in range(3)]

    def wait_coords(b):
        for k in range(3):
            pltpu.make_async_copy(cin[k].at[pl.ds(0, _CH)],
                                  cbuf[b][k], sem_c).wait()

    def compute_idx(b):
        gx, gy, gz = cbuf[b]
        for g in range(_CH // _L):
            sl = pl.ds(g * _L, _L)
            vbuf[b][sl] = _corner_base(gx[sl], gy[sl], gz[sl])

    def fire_gathers(b):
        for j in range(8):
            pltpu.async_copy(tview[j].at[vbuf[b]], rows[b][j], sem_g)

    def wait_gathers(b):
        for j in range(8):
            pltpu.make_async_copy(tview[j].at[vbuf[b]], rows[b][j],
                                  sem_g).wait()

    def blend(b):
        gx, gy, gz = cbuf[b]
        rws = rows[b]
        ob = obuf[b]

        def group(g, c):
            sl = pl.ds(g * _L, _L)
            wx1, wy1, wz1 = _lerp_weights(gx[sl], gy[sl], gz[sl])
            for l in range(_L):
                p = g * _L + l
                wx = wx1[l]
                wy = wy1[l]
                wz = wz1[l]
                c00 = rws[0][p, :]
                c00 = c00 + wx * (rws[1][p, :] - c00)
                c01 = rws[2][p, :]
                c01 = c01 + wx * (rws[3][p, :] - c01)
                c10 = rws[4][p, :]
                c10 = c10 + wx * (rws[5][p, :] - c10)
                c11 = rws[6][p, :]
                c11 = c11 + wx * (rws[7][p, :] - c11)
                c0 = c00 + wy * (c01 - c00)
                c1 = c10 + wy * (c11 - c10)
                ob[p, :] = c0 + wz * (c1 - c0)
            return c

        lax.fori_loop(0, _CH // _L, group, 0, unroll=2)

    def wait_out(b):
        pltpu.make_async_copy(obuf[b], out.at[pl.ds(0, _CH)], sem_o).wait()

    for cp in start_coords(0, 0):
        cp.wait()
    compute_idx(0)
    fire_gathers(0)
    start_coords(1, 1)

    def body(s, carry):
        b = lax.rem(s, 2)

        def side(bb):
            nb = 1 - bb
            wait_coords(nb)
            compute_idx(nb)
            wait_gathers(bb)
            fire_gathers(nb)

            @pl.when(s >= 2)
            def _():
                wait_out(bb)

            blend(bb)
            pltpu.async_copy(obuf[bb], out.at[pl.ds(base_of(s), _CH)], sem_o)

            @pl.when(s < chunks - 2)
            def _():
                start_coords(s + 2, bb)

        @pl.when(b == 0)
        def _():
            side(0)

        @pl.when(b == 1)
        def _():
            side(1)

        return carry

    lax.fori_loop(0, chunks - 1, body, 0)

    lastb = (chunks - 1) % 2
    wait_gathers(lastb)
    wait_out(lastb)
    blend(lastb)
    wait_out(1 - lastb)
    pltpu.sync_copy(obuf[lastb], out.at[pl.ds(base_of(chunks - 1), _CH)])


def kernel(x, data):
    n = x.shape[0]
    table = jnp.transpose(data[0], (1, 2, 3, 0)).reshape(_DHW, _C)
    zero = x[0, 0] * jnp.float32(0.0)
    cx = x[:, 2] + zero
    cy = x[:, 1] + zero
    cz = x[:, 0] + zero

    mesh = plsc.VectorSubcoreMesh(core_axis_name="c", subcore_axis_name="s")
    scratch = (
        [pltpu.VMEM((_CH,), jnp.float32) for _ in range(6)]
        + [pltpu.VMEM((_CH,), jnp.int32) for _ in range(2)]
        + [pltpu.VMEM((_CH, _C), jnp.float32) for _ in range(16)]
        + [pltpu.VMEM((_CH, _C), jnp.float32) for _ in range(2)]
        + [pltpu.SemaphoreType.DMA for _ in range(3)]
    )
    run = functools.partial(
        pl.kernel,
        out_type=jax.ShapeDtypeStruct((n, _C), jnp.float32),
        mesh=mesh,
        scratch_types=scratch,
        compiler_params=pltpu.CompilerParams(use_tc_tiling_on_sc=False),
    )(_sc_body)
    return run(table, cx, cy, cz)


# speedup vs baseline: 1.0019x; 1.0019x over previous
import functools

import jax
import jax.numpy as jnp
from jax import lax
from jax.experimental import pallas as pl
from jax.experimental.pallas import tpu as pltpu
from jax.experimental.pallas import tpu_sc as plsc

_D = _H = _W = 128
_HW = _H * _W
_DHW = _D * _HW
_C = 16
_L = 16
_CH = 128
_NC = 2
_NS = 16
_NW = _NC * _NS
_OFFS = (0, 1, _W, _W + 1, _HW, _HW + 1, _HW + _W, _HW + _W + 1)


def _axis_coords(v, dim):
    t = (v - 0.5) * 2.0
    i = (t + 1.0) * 0.5 * jnp.float32(dim - 1)
    i0 = jnp.minimum(i.astype(jnp.int32), dim - 2)
    w1 = i - i0.astype(jnp.float32)
    return i0, w1, 1.0 - w1


def _corner_base(vx, vy, vz):
    x0, _, _ = _axis_coords(vx, _W)
    y0, _, _ = _axis_coords(vy, _H)
    z0, _, _ = _axis_coords(vz, _D)
    return z0 * _HW + y0 * _W + x0


def _lerp_weights(vx, vy, vz):
    _, wx1, _ = _axis_coords(vx, _W)
    _, wy1, _ = _axis_coords(vy, _H)
    _, wz1, _ = _axis_coords(vz, _D)
    return wx1, wy1, wz1


def _sc_body(table, cx, cy, cz, out, *scr):
    cbuf = (scr[0:3], scr[3:6])
    vbuf = scr[6:8]
    rows = (scr[8:16], scr[16:24])
    obuf = scr[24:26]
    sem_g, sem_c, sem_o = scr[26:29]
    tview = [table.at[pl.ds(o, _DHW - _OFFS[-1])] for o in _OFFS]

    nper = cx.shape[0] // _NW
    chunks = nper // _CH
    wid = lax.axis_index("s") * _NC + lax.axis_index("c")
    cin = (cx, cy, cz)

    def base_of(s):
        return wid * nper + s * _CH

    def start_coords(s, b):
        return [pltpu.async_copy(cin[k].at[pl.ds(base_of(s), _CH)],
                                 cbuf[b][k], sem_c) for k in range(3)]

    def wait_coords(b):
        for k in range(3):
            pltpu.make_async_copy(cin[k].at[pl.ds(0, _CH)],
                                  cbuf[b][k], sem_c).wait()

    def compute_idx(b):
        gx, gy, gz = cbuf[b]
        for g in range(_CH // _L):
            sl = pl.ds(g * _L, _L)
            vbuf[b][sl] = _corner_base(gx[sl], gy[sl], gz[sl])

    def fire_gathers(b):
        for j in range(8):
            pltpu.async_copy(tview[j].at[vbuf[b]], rows[b][j], sem_g)

    def wait_gathers(b):
        for j in range(8):
            pltpu.make_async_copy(tview[j].at[vbuf[b]], rows[b][j],
                                  sem_g).wait()

    def blend(b):
        gx, gy, gz = cbuf[b]
        rws = rows[b]
        ob = obuf[b]

        def group(g, c):
            sl = pl.ds(g * _L, _L)
            wx1, wy1, wz1 = _lerp_weights(gx[sl], gy[sl], gz[sl])
            for l in range(_L):
                p = g * _L + l
                wx = wx1[l]
                wy = wy1[l]
                wz = wz1[l]
                c00 = rws[0][p, :]
                c00 = c00 + wx * (rws[1][p, :] - c00)
                c01 = rws[2][p, :]
                c01 = c01 + wx * (rws[3][p, :] - c01)
                c10 = rws[4][p, :]
                c10 = c10 + wx * (rws[5][p, :] - c10)
                c11 = rws[6][p, :]
                c11 = c11 + wx * (rws[7][p, :] - c11)
                c0 = c00 + wy * (c01 - c00)
                c1 = c10 + wy * (c11 - c10)
                ob[p, :] = c0 + wz * (c1 - c0)
            return c

        lax.fori_loop(0, _CH // _L, group, 0)

    def wait_out(b):
        pltpu.make_async_copy(obuf[b], out.at[pl.ds(0, _CH)], sem_o).wait()

    for cp in start_coords(0, 0):
        cp.wait()
    compute_idx(0)
    fire_gathers(0)
    start_coords(1, 1)

    def body(s, carry):
        b = lax.rem(s, 2)

        def side(bb):
            nb = 1 - bb
            wait_coords(nb)
            compute_idx(nb)
            wait_gathers(bb)
            fire_gathers(nb)

            @pl.when(s >= 2)
            def _():
                wait_out(bb)

            blend(bb)
            pltpu.async_copy(obuf[bb], out.at[pl.ds(base_of(s), _CH)], sem_o)

            @pl.when(s < chunks - 2)
            def _():
                start_coords(s + 2, bb)

        @pl.when(b == 0)
        def _():
            side(0)

        @pl.when(b == 1)
        def _():
            side(1)

        return carry

    lax.fori_loop(0, chunks - 1, body, 0)

    lastb = (chunks - 1) % 2
    wait_gathers(lastb)
    wait_out(lastb)
    blend(lastb)
    wait_out(1 - lastb)
    pltpu.sync_copy(obuf[lastb], out.at[pl.ds(base_of(chunks - 1), _CH)])


def kernel(x, data):
    n = x.shape[0]
    table = jnp.transpose(data[0], (1, 2, 3, 0)).reshape(_DHW, _C)
    cx = x[:, 2]
    cy = x[:, 1]
    cz = x[:, 0]

    mesh = plsc.VectorSubcoreMesh(core_axis_name="c", subcore_axis_name="s")
    scratch = (
        [pltpu.VMEM((_CH,), jnp.float32) for _ in range(6)]
        + [pltpu.VMEM((_CH,), jnp.int32) for _ in range(2)]
        + [pltpu.VMEM((_CH, _C), jnp.float32) for _ in range(16)]
        + [pltpu.VMEM((_CH, _C), jnp.float32) for _ in range(2)]
        + [pltpu.SemaphoreType.DMA for _ in range(3)]
    )
    run = functools.partial(
        pl.kernel,
        out_type=jax.ShapeDtypeStruct((n, _C), jnp.float32),
        mesh=mesh,
        scratch_types=scratch,
        compiler_params=pltpu.CompilerParams(use_tc_tiling_on_sc=False),
    )(_sc_body)
    return run(table, cx, cy, cz)
